# baseline (speedup 1.0000x reference)
import functools

import jax
import jax.numpy as jnp
from jax import lax
from jax.experimental import pallas as pl
from jax.experimental.pallas import tpu as pltpu
from jax.experimental.pallas import tpu_sc as plsc

NUM_PROMPTS = 50
PROMPT_DIM = 768
SEQ = 577
OSEQ = SEQ + NUM_PROMPTS
_CH = 36
_NCHUNK = (SEQ - 1) // _CH
_NBUF = 3


def _sc_splice(layer_hbm, x_hbm, pe_hbm, out_hbm,
               bufs, pbuf, hbuf, lbuf, rsems, wsems, psem):
    info = plsc.get_sparse_core_info()
    nc = info.num_cores
    nw = nc * info.num_subcores
    bsz = x_hbm.shape[0]
    bpw = bsz // nw
    wid = lax.axis_index("s") * nc + lax.axis_index("c")

    pltpu.sync_copy(layer_hbm, lbuf)
    pltpu.make_async_copy(pe_hbm.at[lbuf], pbuf, psem).start()

    def task(t):
        bi, j = divmod(t, _NCHUNK)
        b = wid * bpw + bi
        src = x_hbm.at[b, pl.ds(1 + j * _CH, _CH), :]
        dst = out_hbm.at[b, pl.ds(1 + NUM_PROMPTS + j * _CH, _CH), :]
        return src, dst

    ntask = bpw * _NCHUNK
    for t in range(_NBUF):
        src, _ = task(t)
        pltpu.make_async_copy(src, bufs.at[t % _NBUF], rsems.at[t % _NBUF]).start()

    for t in range(ntask):
        s = t % _NBUF
        src, dst = task(t)
        pltpu.make_async_copy(src, bufs.at[s], rsems.at[s]).wait()
        wcopy = pltpu.make_async_copy(bufs.at[s], dst, wsems.at[s])
        wcopy.start()
        if t + _NBUF < ntask:
            nsrc, _ = task(t + _NBUF)
            wcopy.wait()
            pltpu.make_async_copy(nsrc, bufs.at[s], rsems.at[s]).start()

    pltpu.make_async_copy(pe_hbm.at[lbuf], pbuf, psem).wait()
    for bi in range(bpw):
        b = wid * bpw + bi
        pltpu.sync_copy(x_hbm.at[b, pl.ds(0, 1), :], hbuf)
        pltpu.sync_copy(hbuf, out_hbm.at[b, pl.ds(0, 1), :])
        pltpu.sync_copy(pbuf.at[0], out_hbm.at[b, pl.ds(1, NUM_PROMPTS), :])

    for t in range(max(0, ntask - _NBUF), ntask):
        s = t % _NBUF
        _, dst = task(t)
        pltpu.make_async_copy(bufs.at[s], dst, wsems.at[s]).wait()


def kernel(x, prompt_embeddings, layer_num):
    Bsz = x.shape[0]
    layer = jnp.asarray(layer_num, jnp.int32).reshape((1,))
    mesh = plsc.VectorSubcoreMesh(core_axis_name="c", subcore_axis_name="s")
    run = pl.kernel(
        _sc_splice,
        out_type=jax.ShapeDtypeStruct((Bsz, OSEQ, PROMPT_DIM), jnp.float32),
        mesh=mesh,
        scratch_types=[
            pltpu.VMEM((_NBUF, _CH, PROMPT_DIM), jnp.float32),
            pltpu.VMEM((1, NUM_PROMPTS, PROMPT_DIM), jnp.float32),
            pltpu.VMEM((1, PROMPT_DIM), jnp.float32),
            pltpu.VMEM((1,), jnp.int32),
            pltpu.SemaphoreType.DMA((_NBUF,)),
            pltpu.SemaphoreType.DMA((_NBUF,)),
            pltpu.SemaphoreType.DMA,
        ],
        compiler_params=pltpu.CompilerParams(use_tc_tiling_on_sc=False),
    )
    return run(layer, x, prompt_embeddings)


# speedup vs baseline: 1.3364x; 1.3364x over previous
import jax
import jax.numpy as jnp
from jax import lax
from jax.experimental import pallas as pl
from jax.experimental.pallas import tpu as pltpu
from jax.experimental.pallas import tpu_sc as plsc

NUM_PROMPTS = 50
PROMPT_DIM = 768
SEQ = 577
OSEQ = SEQ + NUM_PROMPTS
_HEAD = 56
_CH = 32
_RB = 40
_NT = 18
_SEG = PROMPT_DIM // 16


def _head_kernel(layer_ref, x_ref, pe_ref, o_ref):
    del layer_ref
    o_ref[0, 0:1, :] = x_ref[0, 0:1, :]
    o_ref[0, 1:1 + NUM_PROMPTS, :] = pe_ref[0]
    o_ref[0, 1 + NUM_PROMPTS:, :] = x_ref[0, 1:_HEAD - NUM_PROMPTS, :]


def _head(x, prompt_embeddings, layer):
    Bsz = x.shape[0]
    grid_spec = pltpu.PrefetchScalarGridSpec(
        num_scalar_prefetch=1,
        grid=(Bsz,),
        in_specs=[
            pl.BlockSpec((1, 8, PROMPT_DIM), lambda b, s: (b, 0, 0)),
            pl.BlockSpec((1, NUM_PROMPTS, PROMPT_DIM),
                         lambda b, s: (s[0], 0, 0)),
        ],
        out_specs=pl.BlockSpec((1, _HEAD, PROMPT_DIM), lambda b, s: (b, 0, 0)),
    )
    return pl.pallas_call(
        _head_kernel,
        grid_spec=grid_spec,
        out_shape=jax.ShapeDtypeStruct((Bsz, _HEAD, PROMPT_DIM), x.dtype),
        compiler_params=pltpu.CompilerParams(
            dimension_semantics=("parallel",)),
    )(layer, x, prompt_embeddings)


def _shift_rows(rbuf, wbuf, n):
    for r in range(n):
        for c in range(_SEG):
            wbuf[0, r, pl.ds(16 * c, 16)] = rbuf[0, 6 + r, pl.ds(16 * c, 16)]


def _tail_kernel(prev_ref, xa_ref, xb_ref, o_ref):
    del prev_ref
    o_ref[0, 0:1, :] = xa_ref[0, 6:7, :]
    o_ref[0, 1:2, :] = xa_ref[0, 7:8, :]
    o_ref[0, 2:3, :] = xb_ref[0, 0:1, :]


def _patch_tail(out_sc, x):
    Bsz = x.shape[0]
    return pl.pallas_call(
        _tail_kernel,
        grid=(Bsz,),
        in_specs=[
            pl.BlockSpec((1, 8, PROMPT_DIM), lambda b: (b, 78, 0)),
            pl.BlockSpec((1, 8, PROMPT_DIM), lambda b: (b, 71, 0)),
            pl.BlockSpec((1, 8, PROMPT_DIM), lambda b: (b, 72, 0)),
        ],
        out_specs=pl.BlockSpec((1, 8, PROMPT_DIM), lambda b: (b, 78, 0)),
        out_shape=jax.ShapeDtypeStruct((Bsz, OSEQ, PROMPT_DIM), x.dtype),
        input_output_aliases={0: 0},
        compiler_params=pltpu.CompilerParams(
            dimension_semantics=("parallel",)),
    )(out_sc, x, x)


def _sc_splice(head_hbm, x_hbm, out_hbm,
               rbuf0, rbuf1, wbuf0, wbuf1, rsems, wsems, hsem):
    info = plsc.get_sparse_core_info()
    nc = info.num_cores
    nw = nc * info.num_subcores
    bsz = x_hbm.shape[0]
    bpw = bsz // nw
    wid = lax.axis_index("s") * nc + lax.axis_index("c")
    rbufs = (rbuf0, rbuf1)
    wbufs = (wbuf0, wbuf1)
    ntask = bpw * _NT

    def rd(t):
        bi = t // _NT
        k = t - bi * _NT
        b = wid * bpw + bi
        w0 = pl.multiple_of(jnp.minimum(_HEAD + _CH * k, 592), 8)
        return (x_hbm.at[pl.ds(b, 1), pl.ds(w0 - _HEAD, _RB), :],
                out_hbm.at[pl.ds(b, 1), pl.ds(w0, _CH), :])

    for bi in range(bpw):
        b = wid * bpw + bi
        wcopies = []
        for s, (o, n) in enumerate(((0, 24), (24, 32))):
            rcopy = pltpu.make_async_copy(
                head_hbm.at[pl.ds(b, 1), pl.ds(o, n), :],
                rbufs[s].at[:, pl.ds(0, n), :], rsems.at[s])
            rcopy.start()
            rcopy.wait()
            wcopy = pltpu.make_async_copy(
                rbufs[s].at[:, pl.ds(0, n), :],
                out_hbm.at[pl.ds(b, 1), pl.ds(o, n), :], hsem)
            wcopy.start()
            wcopies.append(wcopy)
        for wcopy in wcopies:
            wcopy.wait()

    for t in range(2):
        src, _ = rd(t)
        pltpu.make_async_copy(src, rbufs[t].at[:, pl.ds(0, _RB), :],
                              rsems.at[t]).start()

    def body(u, carry):
        for phase in range(2):
            t = 2 * u + phase
            s = phase
            src, dst = rd(t)
            pltpu.make_async_copy(src, rbufs[s].at[:, pl.ds(0, _RB), :],
                                  rsems.at[s]).wait()

            @pl.when(t >= 2)
            def _():
                _, pdst = rd(t - 2)
                pltpu.make_async_copy(wbufs[s].at[:, pl.ds(0, _CH), :],
                                      pdst, wsems.at[s]).wait()

            _shift_rows(rbufs[s], wbufs[s], _CH)
            pltpu.make_async_copy(wbufs[s].at[:, pl.ds(0, _CH), :], dst,
                                  wsems.at[s]).start()

            @pl.when(t + 2 < ntask)
            def _():
                nsrc, _ = rd(t + 2)
                pltpu.make_async_copy(nsrc, rbufs[s].at[:, pl.ds(0, _RB), :],
                                      rsems.at[s]).start()
        return carry

    lax.fori_loop(0, ntask // 2, body, 0)
    for t in range(ntask - 2, ntask):
        s = t % 2
        _, dst = rd(t)
        pltpu.make_async_copy(wbufs[s].at[:, pl.ds(0, _CH), :], dst,
                              wsems.at[s]).wait()


def kernel(x, prompt_embeddings, layer_num):
    Bsz = x.shape[0]
    layer = jnp.asarray(layer_num, jnp.int32).reshape((1,))
    head = _head(x, prompt_embeddings, layer)
    mesh = plsc.VectorSubcoreMesh(core_axis_name="c", subcore_axis_name="s")
    run = pl.kernel(
        _sc_splice,
        out_type=jax.ShapeDtypeStruct((Bsz, OSEQ, PROMPT_DIM), jnp.float32),
        mesh=mesh,
        scratch_types=[
            pltpu.VMEM((1, _RB, PROMPT_DIM), jnp.float32),
            pltpu.VMEM((1, _RB, PROMPT_DIM), jnp.float32),
            pltpu.VMEM((1, _CH, PROMPT_DIM), jnp.float32),
            pltpu.VMEM((1, _CH, PROMPT_DIM), jnp.float32),
            pltpu.SemaphoreType.DMA((2,)),
            pltpu.SemaphoreType.DMA((2,)),
            pltpu.SemaphoreType.DMA,
        ],
    )
    out_sc = run(head, x)
    return _patch_tail(out_sc, x)


# speedup vs baseline: 1.5487x; 1.1589x over previous
import jax
import jax.numpy as jnp
from jax import lax
from jax.experimental import pallas as pl
from jax.experimental.pallas import tpu as pltpu
from jax.experimental.pallas import tpu_sc as plsc

NUM_PROMPTS = 50
PROMPT_DIM = 768
SEQ = 577
OSEQ = SEQ + NUM_PROMPTS
_HEAD = 56
_CH = 32
_RB = 40
_NT = 18
_SEG = PROMPT_DIM // 16


_BB = 8


def _head_kernel(layer_ref, x_ref, pe_ref, o_ref):
    del layer_ref
    o_ref[:, 0:1, :] = x_ref[:, 0:1, :]
    o_ref[:, 1:1 + NUM_PROMPTS, :] = jnp.broadcast_to(
        pe_ref[...], (_BB, NUM_PROMPTS, PROMPT_DIM))
    o_ref[:, 1 + NUM_PROMPTS:, :] = x_ref[:, 1:_HEAD - NUM_PROMPTS, :]


def _head(x, prompt_embeddings, layer):
    Bsz = x.shape[0]
    grid_spec = pltpu.PrefetchScalarGridSpec(
        num_scalar_prefetch=1,
        grid=(Bsz // _BB,),
        in_specs=[
            pl.BlockSpec((_BB, 8, PROMPT_DIM), lambda b, s: (b, 0, 0)),
            pl.BlockSpec((1, NUM_PROMPTS, PROMPT_DIM),
                         lambda b, s: (s[0], 0, 0)),
        ],
        out_specs=pl.BlockSpec((_BB, _HEAD, PROMPT_DIM),
                               lambda b, s: (b, 0, 0)),
    )
    return pl.pallas_call(
        _head_kernel,
        grid_spec=grid_spec,
        out_shape=jax.ShapeDtypeStruct((Bsz, _HEAD, PROMPT_DIM), x.dtype),
        compiler_params=pltpu.CompilerParams(
            dimension_semantics=("parallel",)),
    )(layer, x, prompt_embeddings)


def _shift_rows(rbuf, wbuf, n):
    for r in range(n):
        for c in range(_SEG):
            wbuf[0, r, pl.ds(16 * c, 16)] = rbuf[0, 6 + r, pl.ds(16 * c, 16)]


def _tail_kernel(prev_ref, xa_ref, xb_ref, o_ref):
    del prev_ref
    o_ref[:, 0:1, :] = xa_ref[:, 6:7, :]
    o_ref[:, 1:2, :] = xa_ref[:, 7:8, :]
    o_ref[:, 2:3, :] = xb_ref[:, 0:1, :]


def _patch_tail(out_sc, x):
    Bsz = x.shape[0]
    return pl.pallas_call(
        _tail_kernel,
        grid=(Bsz // _BB,),
        in_specs=[
            pl.BlockSpec((_BB, 8, PROMPT_DIM), lambda b: (b, 78, 0)),
            pl.BlockSpec((_BB, 8, PROMPT_DIM), lambda b: (b, 71, 0)),
            pl.BlockSpec((_BB, 8, PROMPT_DIM), lambda b: (b, 72, 0)),
        ],
        out_specs=pl.BlockSpec((_BB, 8, PROMPT_DIM), lambda b: (b, 78, 0)),
        out_shape=jax.ShapeDtypeStruct((Bsz, OSEQ, PROMPT_DIM), x.dtype),
        input_output_aliases={0: 0},
        compiler_params=pltpu.CompilerParams(
            dimension_semantics=("parallel",)),
    )(out_sc, x, x)


def _sc_splice(head_hbm, x_hbm, out_hbm,
               rbuf0, rbuf1, wbuf0, wbuf1, rsems, wsems, hsem):
    info = plsc.get_sparse_core_info()
    nc = info.num_cores
    nw = nc * info.num_subcores
    bsz = x_hbm.shape[0]
    bpw = bsz // nw
    wid = lax.axis_index("s") * nc + lax.axis_index("c")
    rbufs = (rbuf0, rbuf1)
    wbufs = (wbuf0, wbuf1)
    ntask = bpw * _NT

    def rd(t):
        bi = t // _NT
        k = t - bi * _NT
        b = wid * bpw + bi
        w0 = pl.multiple_of(jnp.minimum(_HEAD + _CH * k, 592), 8)
        return (x_hbm.at[pl.ds(b, 1), pl.ds(w0 - _HEAD, _RB), :],
                out_hbm.at[pl.ds(b, 1), pl.ds(w0, _CH), :])

    for bi in range(bpw):
        b = wid * bpw + bi
        wcopies = []
        for s, (o, n) in enumerate(((0, 24), (24, 32))):
            rcopy = pltpu.make_async_copy(
                head_hbm.at[pl.ds(b, 1), pl.ds(o, n), :],
                rbufs[s].at[:, pl.ds(0, n), :], rsems.at[s])
            rcopy.start()
            rcopy.wait()
            wcopy = pltpu.make_async_copy(
                rbufs[s].at[:, pl.ds(0, n), :],
                out_hbm.at[pl.ds(b, 1), pl.ds(o, n), :], hsem)
            wcopy.start()
            wcopies.append(wcopy)
        for wcopy in wcopies:
            wcopy.wait()

    for t in range(2):
        src, _ = rd(t)
        pltpu.make_async_copy(src, rbufs[t].at[:, pl.ds(0, _RB), :],
                              rsems.at[t]).start()

    def body(u, carry):
        for phase in range(2):
            t = 2 * u + phase
            s = phase
            src, dst = rd(t)
            pltpu.make_async_copy(src, rbufs[s].at[:, pl.ds(0, _RB), :],
                                  rsems.at[s]).wait()

            @pl.when(t >= 2)
            def _():
                _, pdst = rd(t - 2)
                pltpu.make_async_copy(wbufs[s].at[:, pl.ds(0, _CH), :],
                                      pdst, wsems.at[s]).wait()

            _shift_rows(rbufs[s], wbufs[s], _CH)
            pltpu.make_async_copy(wbufs[s].at[:, pl.ds(0, _CH), :], dst,
                                  wsems.at[s]).start()

            @pl.when(t + 2 < ntask)
            def _():
                nsrc, _ = rd(t + 2)
                pltpu.make_async_copy(nsrc, rbufs[s].at[:, pl.ds(0, _RB), :],
                                      rsems.at[s]).start()
        return carry

    lax.fori_loop(0, ntask // 2, body, 0)
    for t in range(ntask - 2, ntask):
        s = t % 2
        _, dst = rd(t)
        pltpu.make_async_copy(wbufs[s].at[:, pl.ds(0, _CH), :], dst,
                              wsems.at[s]).wait()


def kernel(x, prompt_embeddings, layer_num):
    Bsz = x.shape[0]
    layer = jnp.asarray(layer_num, jnp.int32).reshape((1,))
    head = _head(x, prompt_embeddings, layer)
    mesh = plsc.VectorSubcoreMesh(core_axis_name="c", subcore_axis_name="s")
    run = pl.kernel(
        _sc_splice,
        out_type=jax.ShapeDtypeStruct((Bsz, OSEQ, PROMPT_DIM), jnp.float32),
        mesh=mesh,
        scratch_types=[
            pltpu.VMEM((1, _RB, PROMPT_DIM), jnp.float32),
            pltpu.VMEM((1, _RB, PROMPT_DIM), jnp.float32),
            pltpu.VMEM((1, _CH, PROMPT_DIM), jnp.float32),
            pltpu.VMEM((1, _CH, PROMPT_DIM), jnp.float32),
            pltpu.SemaphoreType.DMA((2,)),
            pltpu.SemaphoreType.DMA((2,)),
            pltpu.SemaphoreType.DMA,
        ],
    )
    out_sc = run(head, x)
    return _patch_tail(out_sc, x)


# speedup vs baseline: 1.5785x; 1.0192x over previous
import jax
import jax.numpy as jnp
from jax import lax
from jax.experimental import pallas as pl
from jax.experimental.pallas import tpu as pltpu
from jax.experimental.pallas import tpu_sc as plsc

NUM_PROMPTS = 50
PROMPT_DIM = 768
SEQ = 577
OSEQ = SEQ + NUM_PROMPTS
_HEAD = 56
_CH = 32
_RB = 40
_NT = 18
_SEG = PROMPT_DIM // 16
_BB = 8


def _tail_kernel(xa_ref, xb_ref, o_ref):
    o_ref[:, 0:1, :] = xa_ref[:, 6:7, :]
    o_ref[:, 1:2, :] = xa_ref[:, 7:8, :]
    o_ref[:, 2:8, :] = jnp.broadcast_to(
        xb_ref[:, 0:1, :], (_BB, 6, PROMPT_DIM))


def _tail(x):
    Bsz = x.shape[0]
    return pl.pallas_call(
        _tail_kernel,
        grid=(Bsz // _BB,),
        in_specs=[
            pl.BlockSpec((_BB, 8, PROMPT_DIM), lambda b: (b, 71, 0)),
            pl.BlockSpec((_BB, 8, PROMPT_DIM), lambda b: (b, 72, 0)),
        ],
        out_specs=pl.BlockSpec((_BB, 8, PROMPT_DIM), lambda b: (b, 78, 0)),
        out_shape=jax.ShapeDtypeStruct((Bsz, OSEQ, PROMPT_DIM), x.dtype),
        compiler_params=pltpu.CompilerParams(
            dimension_semantics=("parallel",)),
    )(x, x)


def _head_kernel(layer_ref, prev_ref, x_ref, pe_ref, o_ref):
    del layer_ref, prev_ref
    o_ref[:, 0:1, :] = x_ref[:, 0:1, :]
    o_ref[:, 1:1 + NUM_PROMPTS, :] = jnp.broadcast_to(
        pe_ref[...], (_BB, NUM_PROMPTS, PROMPT_DIM))
    o_ref[:, 1 + NUM_PROMPTS:, :] = x_ref[:, 1:_HEAD - NUM_PROMPTS, :]


def _head(out0, x, prompt_embeddings, layer):
    Bsz = x.shape[0]
    grid_spec = pltpu.PrefetchScalarGridSpec(
        num_scalar_prefetch=1,
        grid=(Bsz // _BB,),
        in_specs=[
            pl.BlockSpec((_BB, _HEAD, PROMPT_DIM), lambda b, s: (b, 0, 0)),
            pl.BlockSpec((_BB, 8, PROMPT_DIM), lambda b, s: (b, 0, 0)),
            pl.BlockSpec((1, NUM_PROMPTS, PROMPT_DIM),
                         lambda b, s: (s[0], 0, 0)),
        ],
        out_specs=pl.BlockSpec((_BB, _HEAD, PROMPT_DIM),
                               lambda b, s: (b, 0, 0)),
    )
    return pl.pallas_call(
        _head_kernel,
        grid_spec=grid_spec,
        out_shape=jax.ShapeDtypeStruct((Bsz, OSEQ, PROMPT_DIM), x.dtype),
        input_output_aliases={1: 0},
        compiler_params=pltpu.CompilerParams(
            dimension_semantics=("parallel",)),
    )(layer, out0, x, prompt_embeddings)


def _shift_rows(rbuf, wbuf, n):
    for r in range(n):
        for c in range(_SEG):
            wbuf[0, r, pl.ds(16 * c, 16)] = rbuf[0, 6 + r, pl.ds(16 * c, 16)]


def _sc_bulk(x_hbm, out_hbm, rbuf0, rbuf1, wbuf0, wbuf1, rsems, wsems):
    info = plsc.get_sparse_core_info()
    nc = info.num_cores
    nw = nc * info.num_subcores
    bsz = x_hbm.shape[0]
    bpw = bsz // nw
    wid = lax.axis_index("s") * nc + lax.axis_index("c")
    rbufs = (rbuf0, rbuf1)
    wbufs = (wbuf0, wbuf1)
    ntask = bpw * _NT

    def rd(t):
        bi = t // _NT
        k = t - bi * _NT
        b = wid * bpw + bi
        w0 = pl.multiple_of(jnp.minimum(_HEAD + _CH * k, 592), 8)
        return (x_hbm.at[pl.ds(b, 1), pl.ds(w0 - _HEAD, _RB), :],
                out_hbm.at[pl.ds(b, 1), pl.ds(w0, _CH), :])

    for t in range(2):
        src, _ = rd(t)
        pltpu.make_async_copy(src, rbufs[t].at[:, pl.ds(0, _RB), :],
                              rsems.at[t]).start()

    def body(u, carry):
        for phase in range(2):
            t = 2 * u + phase
            s = phase
            src, dst = rd(t)
            pltpu.make_async_copy(src, rbufs[s].at[:, pl.ds(0, _RB), :],
                                  rsems.at[s]).wait()

            @pl.when(t >= 2)
            def _():
                _, pdst = rd(t - 2)
                pltpu.make_async_copy(wbufs[s].at[:, pl.ds(0, _CH), :],
                                      pdst, wsems.at[s]).wait()

            _shift_rows(rbufs[s], wbufs[s], _CH)
            pltpu.make_async_copy(wbufs[s].at[:, pl.ds(0, _CH), :], dst,
                                  wsems.at[s]).start()

            @pl.when(t + 2 < ntask)
            def _():
                nsrc, _ = rd(t + 2)
                pltpu.make_async_copy(nsrc, rbufs[s].at[:, pl.ds(0, _RB), :],
                                      rsems.at[s]).start()
        return carry

    lax.fori_loop(0, ntask // 2, body, 0)
    for t in range(ntask - 2, ntask):
        s = t % 2
        _, dst = rd(t)
        pltpu.make_async_copy(wbufs[s].at[:, pl.ds(0, _CH), :], dst,
                              wsems.at[s]).wait()


def kernel(x, prompt_embeddings, layer_num):
    layer = jnp.asarray(layer_num, jnp.int32).reshape((1,))
    out0 = _head(_tail(x), x, prompt_embeddings, layer)
    mesh = plsc.VectorSubcoreMesh(core_axis_name="c", subcore_axis_name="s")
    run = pl.kernel(
        _sc_bulk,
        out_type=(),
        mesh=mesh,
        scratch_types=[
            pltpu.VMEM((1, _RB, PROMPT_DIM), jnp.float32),
            pltpu.VMEM((1, _RB, PROMPT_DIM), jnp.float32),
            pltpu.VMEM((1, _CH, PROMPT_DIM), jnp.float32),
            pltpu.VMEM((1, _CH, PROMPT_DIM), jnp.float32),
            pltpu.SemaphoreType.DMA((2,)),
            pltpu.SemaphoreType.DMA((2,)),
        ],
        compiler_params=pltpu.CompilerParams(use_tc_tiling_on_sc=True),
    )
    out_ref = jax.new_ref(out0)
    run(x, out_ref)
    return jax.freeze(out_ref)


# speedup vs baseline: 1.5942x; 1.0099x over previous
import jax
import jax.numpy as jnp
from jax import lax
from jax.experimental import pallas as pl
from jax.experimental.pallas import tpu as pltpu
from jax.experimental.pallas import tpu_sc as plsc

NUM_PROMPTS = 50
PROMPT_DIM = 768
SEQ = 577
OSEQ = SEQ + NUM_PROMPTS
_HEAD = 56
_CH = 16
_RB = 24
_NT = 36
_NS = 3
_SEG = PROMPT_DIM // 16
_BB = 8


def _tail_kernel(xa_ref, xb_ref, o_ref):
    o_ref[:, 0:1, :] = xa_ref[:, 6:7, :]
    o_ref[:, 1:2, :] = xa_ref[:, 7:8, :]
    o_ref[:, 2:8, :] = jnp.broadcast_to(
        xb_ref[:, 0:1, :], (_BB, 6, PROMPT_DIM))


def _tail(x):
    Bsz = x.shape[0]
    return pl.pallas_call(
        _tail_kernel,
        grid=(Bsz // _BB,),
        in_specs=[
            pl.BlockSpec((_BB, 8, PROMPT_DIM), lambda b: (b, 71, 0)),
            pl.BlockSpec((_BB, 8, PROMPT_DIM), lambda b: (b, 72, 0)),
        ],
        out_specs=pl.BlockSpec((_BB, 8, PROMPT_DIM), lambda b: (b, 78, 0)),
        out_shape=jax.ShapeDtypeStruct((Bsz, OSEQ, PROMPT_DIM), x.dtype),
        compiler_params=pltpu.CompilerParams(
            dimension_semantics=("parallel",)),
    )(x, x)


def _head_kernel(layer_ref, prev_ref, x_ref, pe_ref, o_ref):
    del layer_ref, prev_ref
    o_ref[:, 0:1, :] = x_ref[:, 0:1, :]
    o_ref[:, 1:1 + NUM_PROMPTS, :] = jnp.broadcast_to(
        pe_ref[...], (_BB, NUM_PROMPTS, PROMPT_DIM))
    o_ref[:, 1 + NUM_PROMPTS:, :] = x_ref[:, 1:_HEAD - NUM_PROMPTS, :]


def _head(out0, x, prompt_embeddings, layer):
    Bsz = x.shape[0]
    grid_spec = pltpu.PrefetchScalarGridSpec(
        num_scalar_prefetch=1,
        grid=(Bsz // _BB,),
        in_specs=[
            pl.BlockSpec((_BB, _HEAD, PROMPT_DIM), lambda b, s: (b, 0, 0)),
            pl.BlockSpec((_BB, 8, PROMPT_DIM), lambda b, s: (b, 0, 0)),
            pl.BlockSpec((1, NUM_PROMPTS, PROMPT_DIM),
                         lambda b, s: (s[0], 0, 0)),
        ],
        out_specs=pl.BlockSpec((_BB, _HEAD, PROMPT_DIM),
                               lambda b, s: (b, 0, 0)),
    )
    return pl.pallas_call(
        _head_kernel,
        grid_spec=grid_spec,
        out_shape=jax.ShapeDtypeStruct((Bsz, OSEQ, PROMPT_DIM), x.dtype),
        input_output_aliases={1: 0},
        compiler_params=pltpu.CompilerParams(
            dimension_semantics=("parallel",)),
    )(layer, out0, x, prompt_embeddings)


def _shift_rows(rbuf, wbuf, n):
    for r in range(n):
        for c in range(_SEG):
            wbuf[0, r, pl.ds(16 * c, 16)] = rbuf[0, 6 + r, pl.ds(16 * c, 16)]


def _sc_bulk(x_hbm, out_hbm, rbuf0, rbuf1, rbuf2, wbuf0, wbuf1, wbuf2,
             rsems, wsems):
    info = plsc.get_sparse_core_info()
    nc = info.num_cores
    nw = nc * info.num_subcores
    bsz = x_hbm.shape[0]
    bpw = bsz // nw
    wid = lax.axis_index("s") * nc + lax.axis_index("c")
    rbufs = (rbuf0, rbuf1, rbuf2)
    wbufs = (wbuf0, wbuf1, wbuf2)
    ntask = bpw * _NT

    def rd(t):
        bi = t // _NT
        k = t - bi * _NT
        b = wid * bpw + bi
        w0 = pl.multiple_of(jnp.minimum(_HEAD + _CH * k, 608), 8)
        return (x_hbm.at[pl.ds(b, 1), pl.ds(w0 - _HEAD, _RB), :],
                out_hbm.at[pl.ds(b, 1), pl.ds(w0, _CH), :])

    for t in range(_NS):
        src, _ = rd(t)
        pltpu.make_async_copy(src, rbufs[t].at[:, pl.ds(0, _RB), :],
                              rsems.at[t]).start()

    def body(u, carry):
        for phase in range(_NS):
            t = _NS * u + phase
            s = phase
            src, dst = rd(t)
            pltpu.make_async_copy(src, rbufs[s].at[:, pl.ds(0, _RB), :],
                                  rsems.at[s]).wait()

            @pl.when(t >= _NS)
            def _():
                _, pdst = rd(t - _NS)
                pltpu.make_async_copy(wbufs[s].at[:, pl.ds(0, _CH), :],
                                      pdst, wsems.at[s]).wait()

            _shift_rows(rbufs[s], wbufs[s], _CH)
            pltpu.make_async_copy(wbufs[s].at[:, pl.ds(0, _CH), :], dst,
                                  wsems.at[s]).start()

            @pl.when(t + _NS < ntask)
            def _():
                nsrc, _ = rd(t + _NS)
                pltpu.make_async_copy(nsrc, rbufs[s].at[:, pl.ds(0, _RB), :],
                                      rsems.at[s]).start()
        return carry

    lax.fori_loop(0, ntask // _NS, body, 0)
    for t in range(ntask - _NS, ntask):
        s = t % _NS
        _, dst = rd(t)
        pltpu.make_async_copy(wbufs[s].at[:, pl.ds(0, _CH), :], dst,
                              wsems.at[s]).wait()


def kernel(x, prompt_embeddings, layer_num):
    layer = jnp.asarray(layer_num, jnp.int32).reshape((1,))
    out0 = _head(_tail(x), x, prompt_embeddings, layer)
    mesh = plsc.VectorSubcoreMesh(core_axis_name="c", subcore_axis_name="s")
    run = pl.kernel(
        _sc_bulk,
        out_type=(),
        mesh=mesh,
        scratch_types=[
            pltpu.VMEM((1, _RB, PROMPT_DIM), jnp.float32),
            pltpu.VMEM((1, _RB, PROMPT_DIM), jnp.float32),
            pltpu.VMEM((1, _RB, PROMPT_DIM), jnp.float32),
            pltpu.VMEM((1, _CH, PROMPT_DIM), jnp.float32),
            pltpu.VMEM((1, _CH, PROMPT_DIM), jnp.float32),
            pltpu.VMEM((1, _CH, PROMPT_DIM), jnp.float32),
            pltpu.SemaphoreType.DMA((_NS,)),
            pltpu.SemaphoreType.DMA((_NS,)),
        ],
        compiler_params=pltpu.CompilerParams(use_tc_tiling_on_sc=True),
    )
    out_ref = jax.new_ref(out0)
    run(x, out_ref)
    return jax.freeze(out_ref)
